# baseline (speedup 1.0000x reference)
import functools

import jax
import jax.numpy as jnp
from jax import lax
from jax.experimental import pallas as pl
from jax.experimental.pallas import tpu as pltpu
from jax.experimental.pallas import tpu_sc as plsc

_N = 16384

_mesh = plsc.ScalarSubcoreMesh(axis_name="c", num_cores=1)


_H = _N // 2


@functools.partial(
    pl.kernel,
    mesh=_mesh,
    out_type=jax.ShapeDtypeStruct((_N,), jnp.float32),
    scratch_types=[pltpu.SemaphoreType.DMA, pltpu.SemaphoreType.DMA],
)
def _copy_kernel(a_hbm, out_hbm, sem0, sem1):
    c0 = pltpu.async_copy(a_hbm.at[pl.ds(0, _H)], out_hbm.at[pl.ds(0, _H)], sem0)
    c1 = pltpu.async_copy(a_hbm.at[pl.ds(_H, _H)], out_hbm.at[pl.ds(_H, _H)], sem1)
    c0.wait()
    c1.wait()


def kernel(a):
    return _copy_kernel(a)
